# baseline (speedup 1.0000x reference)
import functools

import jax
import jax.numpy as jnp
from jax import lax
from jax.experimental import pallas as pl
from jax.experimental.pallas import tpu as pltpu
from jax.experimental.pallas import tpu_sc as plsc

NC = 2
NS = 16
NW = NC * NS
L = 16
CH = 64


def kernel(attr_map, weight_u, weight_v, u_low, v_low, u_high, v_high):
    B, H, W, C = attr_map.shape
    N = u_low.shape[0]
    PW = -(-N // (NW * CH)) * CH
    Npad = NW * PW

    table = lax.optimization_barrier(attr_map.reshape(B, H * W, C))
    wu = weight_u.reshape(N)
    wv = weight_v.reshape(N)
    if Npad != N:
        pad = Npad - N
        zi = jnp.zeros((pad,), jnp.int32)
        zf = jnp.zeros((pad,), jnp.float32)
        ul = jnp.concatenate([u_low, zi])
        vl = jnp.concatenate([v_low, zi])
        uh = jnp.concatenate([u_high, zi])
        vh = jnp.concatenate([v_high, zi])
        wu = jnp.concatenate([wu, zf])
        wv = jnp.concatenate([wv, zf])
    else:
        ul, vl, uh, vh = u_low, v_low, u_high, v_high

    def body(table_h, ul_h, vl_h, uh_h, vh_h, wu_h, wv_h, out_h,
             ul_v, vl_v, uh_v, vh_v, wu_v, s2_v, s3_v,
             i00, i01, i11, s00, s01, s11,
             g00, g01, g11, o_v, gsem, osem):
        w = lax.axis_index("s") * NC + lax.axis_index("c")
        nbase = w * PW
        npts = jnp.minimum(PW, N - nbase)
        nchunks = npts // CH

        pltpu.sync_copy(ul_h.at[pl.ds(nbase, PW)], ul_v)
        pltpu.sync_copy(vl_h.at[pl.ds(nbase, PW)], vl_v)
        pltpu.sync_copy(uh_h.at[pl.ds(nbase, PW)], uh_v)
        pltpu.sync_copy(vh_h.at[pl.ds(nbase, PW)], vh_v)
        pltpu.sync_copy(wu_h.at[pl.ds(nbase, PW)], wu_v)
        pltpu.sync_copy(wv_h.at[pl.ds(nbase, PW)], s3_v)

        def prep(t, _):
            sl = pl.ds(t * L, L)
            ulv = ul_v[sl]
            vlv = vl_v[sl]
            uhv = uh_v[sl]
            vhv = vh_v[sl]
            i00[sl] = vlv * W + ulv
            i01[sl] = vhv * W + ulv
            i11[sl] = vhv * W + uhv
            wuv = wu_v[sl]
            wvv = s3_v[sl]
            t1 = 1.0 - wuv
            p2 = t1 * wvv
            s2_v[sl] = p2
            s3_v[sl] = t1 - p2
            return 0
        lax.fori_loop(0, PW // L, prep, 0)

        def drain_g(tb):
            pltpu.make_async_copy(tb.at[s00.at[0]], g00.at[0], gsem).wait()

        def drain_o(ob):
            pltpu.make_async_copy(o_v.at[0], ob.at[pl.ds(0, CH)], osem).wait()

        def stage_and_fire(tb, j):
            sl = j % 3
            off = j * CH
            for k in range(CH // L):
                sl_d = pl.ds(k * L, L)
                sl_s = pl.ds(off + k * L, L)
                s00[sl, sl_d] = i00[sl_s]
                s01[sl, sl_d] = i01[sl_s]
                s11[sl, sl_d] = i11[sl_s]
            pltpu.async_copy(tb.at[s00.at[sl]], g00.at[sl], gsem)
            pltpu.async_copy(tb.at[s01.at[sl]], g01.at[sl], gsem)
            pltpu.async_copy(tb.at[s11.at[sl]], g11.at[sl], gsem)

        for b in range(B):
            tb = table_h.at[b]
            ob = out_h.at[b]
            stage_and_fire(tb, 0)

            @pl.when(nchunks >= 2)
            def _():
                stage_and_fire(tb, 1)

            def chunk_body(j, _):
                sl = j % 3
                off = j * CH
                drain_g(tb)
                drain_g(tb)
                drain_g(tb)

                @pl.when(j + 2 < nchunks)
                def _():
                    stage_and_fire(tb, j + 2)

                so = j & 1

                @pl.when(j >= 2)
                def _():
                    drain_o(ob)

                def grp_body(q, _):
                    gb = off + q * L
                    a1v = wu_v[pl.ds(gb, L)]
                    a2v = s2_v[pl.ds(gb, L)]
                    a3v = s3_v[pl.ds(gb, L)]
                    for t in range(L):
                        p = q * L + t
                        a1 = a1v[t]
                        a2 = a2v[t]
                        a3 = a3v[t]
                        for c in range(C // L):
                            cs = pl.ds(c * L, L)
                            o_v[so, p, cs] = (a1 * g11[sl, p, cs]
                                              + a2 * g01[sl, p, cs]
                                              + a3 * g00[sl, p, cs])
                    return 0
                lax.fori_loop(0, CH // L, grp_body, 0)

                pltpu.async_copy(o_v.at[so], ob.at[pl.ds(nbase + off, CH)],
                                 osem)
                return 0
            lax.fori_loop(0, nchunks, chunk_body, 0)

            @pl.when(nchunks >= 1)
            def _():
                drain_o(ob)

            @pl.when(nchunks >= 2)
            def _():
                drain_o(ob)

    mesh = plsc.VectorSubcoreMesh(core_axis_name="c", subcore_axis_name="s",
                                  num_cores=NC, num_subcores=NS)
    f = pl.kernel(
        body,
        out_type=jax.ShapeDtypeStruct((B, N, C), jnp.float32),
        mesh=mesh,
        compiler_params=pltpu.CompilerParams(use_tc_tiling_on_sc=False),
        scratch_types=[
            pltpu.VMEM((PW,), jnp.int32),
            pltpu.VMEM((PW,), jnp.int32),
            pltpu.VMEM((PW,), jnp.int32),
            pltpu.VMEM((PW,), jnp.int32),
            pltpu.VMEM((PW,), jnp.float32),
            pltpu.VMEM((PW,), jnp.float32),
            pltpu.VMEM((PW,), jnp.float32),
            pltpu.VMEM((PW,), jnp.int32),
            pltpu.VMEM((PW,), jnp.int32),
            pltpu.VMEM((PW,), jnp.int32),
            pltpu.VMEM((3, CH), jnp.int32),
            pltpu.VMEM((3, CH), jnp.int32),
            pltpu.VMEM((3, CH), jnp.int32),
            pltpu.VMEM((3, CH, C), jnp.float32),
            pltpu.VMEM((3, CH, C), jnp.float32),
            pltpu.VMEM((3, CH, C), jnp.float32),
            pltpu.VMEM((2, CH, C), jnp.float32),
            pltpu.SemaphoreType.DMA,
            pltpu.SemaphoreType.DMA,
        ],
    )
    return f(table, ul, vl, uh, vh, wu, wv)
